# baseline (speedup 1.0000x reference)
import functools

import jax
import jax.numpy as jnp
from jax import lax
from jax.experimental import pallas as pl
from jax.experimental.pallas import tpu as pltpu
from jax.experimental.pallas import tpu_sc as plsc

NC = 2
NS = 16
NW = NC * NS

CL = 512
NBUF = 2
SEG = 2048
IDXC = 2048


def _make_k1(B, V, D):
    LP = ((V + 127) // 128) * 128
    QTOT = LP // 128
    QW = -(-QTOT // NW)
    LW = QW * 128
    NCH = -(-QW * 128 // CL)
    mesh = plsc.VectorSubcoreMesh(core_axis_name="c", subcore_axis_name="s")

    @functools.partial(
        pl.kernel,
        mesh=mesh,
        out_type=jax.ShapeDtypeStruct((B + 128, 128), jnp.float32),
        scratch_types=[
            pltpu.VMEM((NBUF, D, CL), jnp.float32),
            pltpu.VMEM((IDXC,), jnp.int32),
            pltpu.VMEM((B + 16,), jnp.int32),
            pltpu.VMEM((B + 16,), jnp.int32),
            pltpu.VMEM((SEG + 16,), jnp.int32),
            pltpu.VMEM((SEG + 16,), jnp.int32),
            pltpu.VMEM((128, 128), jnp.float32),
            pltpu.VMEM((1, 128), jnp.int32),
            pltpu.SemaphoreType.DMA,
            pltpu.SemaphoreType.DMA,
            pltpu.SemaphoreType.DMA,
            pltpu.SemaphoreType.DMA,
            pltpu.SemaphoreType.DMA,
        ],
        compiler_params=pltpu.CompilerParams(
            use_tc_tiling_on_sc=True, needs_layout_passes=False
        ),
    )
    def k1(idx_hbm, table_hbm, out_hbm,
           stage_v, idxb_v, il_v, bl_v, cil_v, cbl_v, ob_v, idx1_v,
           sem0, sem1, sem2, sem3, semsc):

        def flush_block(n_valid):
            zero = jnp.broadcast_to(jnp.int32(0), (16,))
            c64 = jnp.broadcast_to(jnp.int32(64), (16,))
            for g in range(8):
                lanes = g * 16 + lax.iota(jnp.int32, 16)
                bf = plsc.load_gather(ob_v, [lanes, c64])
                bi = plsc.bitcast(bf, jnp.int32)
                sel = jnp.where(lanes < n_valid, bi, B + lanes)
                plsc.store_scatter(idx1_v, [zero, lanes], sel)
            pltpu.async_copy(ob_v, out_hbm.at[idx1_v.at[0]], semsc).wait()
        iota16 = lax.iota(jnp.int32, 16)
        wid = lax.axis_index("s") * NC + lax.axis_index("c")
        lo = wid * LW
        hi = jnp.minimum(lo + LW, V)
        sems = (sem0, sem1, sem2, sem3)

        def st_off(c):
            return jnp.minimum(lo + CL * c, LP - CL)

        def start_chunk(c, par):
            soff = st_off(c)
            for a in range(D // 8):
                pltpu.async_copy(
                    table_hbm.at[pl.ds(8 * a, 8), pl.ds(soff, CL)],
                    stage_v.at[par, pl.ds(8 * a, 8)], sems[par])

        for b in range(NBUF):
            start_chunk(jnp.int32(b), b)

        def passa_t(t, off):
            pltpu.sync_copy(idx_hbm.at[pl.ds(t * IDXC, IDXC)], idxb_v)

            def passa_u(u, off):
                v = idxb_v[pl.ds(u * 16, 16)]
                m = (v >= lo) & (v < hi)
                n = plsc.all_reduce_population_count(m)[0]

                @pl.when(n > 0)
                def _():
                    plsc.store_compressed(il_v.at[pl.ds(off, 16)], v, mask=m)
                    bvec = t * IDXC + u * 16 + iota16
                    plsc.store_compressed(
                        bl_v.at[pl.ds(off, 16)], bvec, mask=m)

                return off + n

            return lax.fori_loop(0, IDXC // 16, passa_u, off)

        n_w = lax.fori_loop(0, B // IDXC, passa_t, jnp.int32(0))

        NCH_E = -(-NCH // NBUF) * NBUF

        def do_chunk(c, par, slot):
            pltpu.make_async_copy(
                table_hbm.at[:, pl.ds(st_off(0), CL)],
                stage_v.at[par], sems[par]).wait()
            lane0 = lo + CL * c
            soff = st_off(c)
            chunk_hi = jnp.minimum(lane0 + CL, hi)

            def seg_body(s, slot):
                seg_n = jnp.minimum(jnp.int32(SEG), n_w - s * SEG)

                def rescan(u, moff):
                    e0 = s * SEG + u * 16
                    iv = il_v[pl.ds(e0, 16)]
                    bv = bl_v[pl.ds(e0, 16)]
                    ev = (e0 + iota16) < n_w
                    m = (iv >= lane0) & (iv < chunk_hi) & ev
                    mi = m.astype(jnp.int32)
                    cs = plsc.cumsum(mi)
                    pos = moff + cs - mi
                    plsc.store_scatter(cil_v, [pos], iv, mask=m)
                    plsc.store_scatter(cbl_v, [pos], bv, mask=m)
                    return moff + plsc.all_reduce_population_count(m)[0]

                m_cnt = lax.fori_loop(
                    0, (seg_n + 15) // 16, rescan, jnp.int32(0))

                def entry(e, slot):
                    i_s = cil_v[pl.ds(e, 16)][0]
                    b_s = cbl_v[pl.ds(e, 16)][0]
                    cloc = i_s - soff
                    lanes = jnp.broadcast_to(cloc, (16,))
                    pars = jnp.broadcast_to(jnp.int32(par), (16,))
                    srow = lax.rem(slot, 128)
                    rowv = jnp.broadcast_to(srow, (16,))
                    for g in range(D // 16):
                        rows = g * 16 + iota16
                        vals = plsc.load_gather(stage_v, [pars, rows, lanes])
                        plsc.store_scatter(ob_v, [rowv, rows], vals)
                    bvecf = plsc.bitcast(
                        jnp.broadcast_to(b_s, (16,)), jnp.float32)
                    plsc.store_scatter(ob_v, [rowv, 64 + iota16], bvecf)

                    @pl.when(srow == 127)
                    def _():
                        flush_block(jnp.int32(128))

                    return slot + 1

                return lax.fori_loop(0, m_cnt, entry, slot)

            slot = lax.fori_loop(0, (n_w + SEG - 1) // SEG, seg_body, slot)

            @pl.when(c + NBUF < NCH_E)
            def _():
                start_chunk(c + NBUF, par)

            return slot

        def ring_body(p, slot):
            for b in range(NBUF):
                slot = do_chunk(NBUF * p + b, b, slot)
            return slot

        slot = lax.fori_loop(0, NCH_E // NBUF, ring_body, jnp.int32(0))

        @pl.when(lax.rem(slot, 128) != 0)
        def _():
            flush_block(lax.rem(slot, 128))

    return k1


def kernel(cond, embedding_table):
    B, = cond.shape
    V, D = embedding_table.shape
    idx = cond.astype(jnp.int32)
    out_pad = _make_k1(B, V, D)(idx, embedding_table.T)
    return out_pad[:B, :D]


# speedup vs baseline: 1.0188x; 1.0188x over previous
import functools

import jax
import jax.numpy as jnp
from jax import lax
from jax.experimental import pallas as pl
from jax.experimental.pallas import tpu as pltpu
from jax.experimental.pallas import tpu_sc as plsc

NC = 2
NS = 16
NW = NC * NS

CL = 512
NBUF = 2
SEG = 2048
IDXC = 8192


def _make_k1(B, V, D):
    LP = ((V + 127) // 128) * 128
    QTOT = LP // 128
    QW = -(-QTOT // NW)
    LW = QW * 128
    NCH = -(-QW * 128 // CL)
    mesh = plsc.VectorSubcoreMesh(core_axis_name="c", subcore_axis_name="s")

    @functools.partial(
        pl.kernel,
        mesh=mesh,
        out_type=jax.ShapeDtypeStruct((B + 128, 128), jnp.float32),
        scratch_types=[
            pltpu.VMEM((NBUF, D, CL), jnp.float32),
            pltpu.VMEM((IDXC,), jnp.int32),
            pltpu.VMEM((B + 16,), jnp.int32),
            pltpu.VMEM((B + 16,), jnp.int32),
            pltpu.VMEM((SEG + 16,), jnp.int32),
            pltpu.VMEM((SEG + 16,), jnp.int32),
            pltpu.VMEM((128, 128), jnp.float32),
            pltpu.VMEM((1, 128), jnp.int32),
            pltpu.SemaphoreType.DMA,
            pltpu.SemaphoreType.DMA,
            pltpu.SemaphoreType.DMA,
            pltpu.SemaphoreType.DMA,
            pltpu.SemaphoreType.DMA,
        ],
        compiler_params=pltpu.CompilerParams(
            use_tc_tiling_on_sc=True, needs_layout_passes=False
        ),
    )
    def k1(idx_hbm, table_hbm, out_hbm,
           stage_v, idxb_v, il_v, bl_v, cil_v, cbl_v, ob_v, idx1_v,
           sem0, sem1, sem2, sem3, semsc):

        def flush_block(n_valid):
            zero = jnp.broadcast_to(jnp.int32(0), (16,))
            c64 = jnp.broadcast_to(jnp.int32(64), (16,))
            for g in range(8):
                lanes = g * 16 + lax.iota(jnp.int32, 16)
                bf = plsc.load_gather(ob_v, [lanes, c64])
                bi = plsc.bitcast(bf, jnp.int32)
                sel = jnp.where(lanes < n_valid, bi, B + lanes)
                plsc.store_scatter(idx1_v, [zero, lanes], sel)
            pltpu.async_copy(ob_v, out_hbm.at[idx1_v.at[0]], semsc).wait()
        iota16 = lax.iota(jnp.int32, 16)
        wid = lax.axis_index("s") * NC + lax.axis_index("c")
        lo = wid * LW
        hi = jnp.minimum(lo + LW, V)
        sems = (sem0, sem1, sem2, sem3)

        def st_off(c):
            return jnp.minimum(lo + CL * c, LP - CL)

        for b in range(NBUF):
            pltpu.async_copy(table_hbm.at[:, pl.ds(st_off(b), CL)],
                             stage_v.at[b], sems[b])

        def passa_t(t, off):
            pltpu.sync_copy(idx_hbm.at[pl.ds(t * IDXC, IDXC)], idxb_v)

            def passa_u(u, off):
                v = idxb_v[pl.ds(u * 16, 16)]
                m = (v >= lo) & (v < hi)
                mi = m.astype(jnp.int32)
                cs = plsc.cumsum(mi)
                pos = off + cs - mi
                plsc.store_scatter(il_v, [pos], v, mask=m)
                bvec = t * IDXC + u * 16 + iota16
                plsc.store_scatter(bl_v, [pos], bvec, mask=m)
                return off + plsc.all_reduce_population_count(m)[0]

            return lax.fori_loop(0, IDXC // 16, passa_u, off)

        n_w = lax.fori_loop(0, B // IDXC, passa_t, jnp.int32(0))

        NCH_E = -(-NCH // NBUF) * NBUF

        def do_chunk(c, par, slot):
            pltpu.make_async_copy(
                table_hbm.at[:, pl.ds(st_off(0), CL)],
                stage_v.at[par], sems[par]).wait()
            lane0 = lo + CL * c
            soff = st_off(c)
            chunk_hi = jnp.minimum(lane0 + CL, hi)

            def seg_body(s, slot):
                seg_n = jnp.minimum(jnp.int32(SEG), n_w - s * SEG)

                def rescan(u, moff):
                    e0 = s * SEG + u * 16
                    iv = il_v[pl.ds(e0, 16)]
                    bv = bl_v[pl.ds(e0, 16)]
                    ev = (e0 + iota16) < n_w
                    m = (iv >= lane0) & (iv < chunk_hi) & ev
                    mi = m.astype(jnp.int32)
                    cs = plsc.cumsum(mi)
                    pos = moff + cs - mi
                    plsc.store_scatter(cil_v, [pos], iv, mask=m)
                    plsc.store_scatter(cbl_v, [pos], bv, mask=m)
                    return moff + plsc.all_reduce_population_count(m)[0]

                m_cnt = lax.fori_loop(
                    0, (seg_n + 15) // 16, rescan, jnp.int32(0))

                def entry(e, slot):
                    i_s = cil_v[pl.ds(e, 16)][0]
                    b_s = cbl_v[pl.ds(e, 16)][0]
                    cloc = i_s - soff
                    lanes = jnp.broadcast_to(cloc, (16,))
                    pars = jnp.broadcast_to(jnp.int32(par), (16,))
                    srow = lax.rem(slot, 128)
                    rowv = jnp.broadcast_to(srow, (16,))
                    for g in range(D // 16):
                        rows = g * 16 + iota16
                        vals = plsc.load_gather(stage_v, [pars, rows, lanes])
                        plsc.store_scatter(ob_v, [rowv, rows], vals)
                    bvecf = plsc.bitcast(
                        jnp.broadcast_to(b_s, (16,)), jnp.float32)
                    plsc.store_scatter(ob_v, [rowv, 64 + iota16], bvecf)

                    @pl.when(srow == 127)
                    def _():
                        flush_block(jnp.int32(128))

                    return slot + 1

                return lax.fori_loop(0, m_cnt, entry, slot)

            slot = lax.fori_loop(0, (n_w + SEG - 1) // SEG, seg_body, slot)

            @pl.when(c + NBUF < NCH_E)
            def _():
                pltpu.async_copy(
                    table_hbm.at[:, pl.ds(st_off(c + NBUF), CL)],
                    stage_v.at[par], sems[par])

            return slot

        def ring_body(p, slot):
            for b in range(NBUF):
                slot = do_chunk(NBUF * p + b, b, slot)
            return slot

        slot = lax.fori_loop(0, NCH_E // NBUF, ring_body, jnp.int32(0))

        @pl.when(lax.rem(slot, 128) != 0)
        def _():
            flush_block(lax.rem(slot, 128))

    return k1


def kernel(cond, embedding_table):
    B, = cond.shape
    V, D = embedding_table.shape
    idx = cond.astype(jnp.int32)
    out_pad = _make_k1(B, V, D)(idx, embedding_table.T)
    return out_pad[:B, :D]
